# Initial kernel scaffold; baseline (speedup 1.0000x reference)
import jax
import jax.numpy as jnp
from jax.experimental import pallas as pl


def kernel(x):
    raise NotImplementedError("write your pallas kernel here")


# speedup vs baseline: 1.7264x; 1.7264x over previous
import numpy as np
import jax
import jax.numpy as jnp
from jax.experimental import pallas as pl
from jax.experimental.pallas import tpu as pltpu

_RATIO = 2


def _pooled_size(n: int) -> int:
    return (n - 1) // 2 + 1


def _pool_matrix_np(in_size: int) -> np.ndarray:
    out_size = _pooled_size(in_size)
    i = np.arange(out_size)[:, None]
    h = np.arange(in_size)[None, :]
    return ((h >= 2 * i - 1) & (h <= 2 * i + 1)).astype(np.float32)


def _composed_axis_mats(H: int, W: int, ratio: int):
    hc = None
    wc = None
    h, w = H, W
    for _ in range(ratio):
        hm = _pool_matrix_np(h)
        wm = _pool_matrix_np(w)
        hc = hm if hc is None else (hm @ hc)
        wc = wm if wc is None else (wm @ wc)
        h, w = hm.shape[0], wm.shape[0]
    wct = (wc.T / np.float32(9.0 ** ratio)).astype(np.float32)
    return hc.astype(np.float32), wct, h, w


def _pool_kernel(x_ref, wt_ref, hc_ref, o_ref):
    tb, H, W = x_ref.shape
    wo = wt_ref.shape[1]
    ho = hc_ref.shape[0]
    x = x_ref[...]
    y = jnp.dot(x.reshape(tb * H, W), wt_ref[...],
                preferred_element_type=jnp.float32).reshape(tb, H, wo)
    hc_b = jnp.broadcast_to(hc_ref[...], (tb, ho, H))
    z = jnp.einsum('bip,bpj->bij', hc_b, y,
                   preferred_element_type=jnp.float32)
    o_ref[...] = z.astype(o_ref.dtype)


def _cdiv(a: int, b: int) -> int:
    return -(-a // b)


def kernel(x):
    N, C, H, W = x.shape
    B = N * C
    hc_np, wct_np, ho, wo = _composed_axis_mats(H, W, _RATIO)

    tb = 8
    while tb > 1 and (B % tb or _cdiv(B, tb) % 2):
        tb //= 2
    steps = _cdiv(B, tb)
    B_pad = steps * tb

    xb = x.reshape(B, H, W)
    if B_pad > B:
        xb = jnp.pad(xb, ((0, B_pad - B), (0, 0), (0, 0)))

    flops = 2 * B_pad * (H * W * wo + ho * H * wo)
    bytes_accessed = (B_pad * (H * W + ho * wo) * 4
                      + (hc_np.size + wct_np.size) * 4)

    out = pl.pallas_call(
        _pool_kernel,
        out_shape=jax.ShapeDtypeStruct((B_pad, ho, wo), x.dtype),
        grid=(steps,),
        in_specs=[pl.BlockSpec((tb, H, W), lambda b: (b, 0, 0)),
                  pl.BlockSpec(wct_np.shape, lambda b: (0, 0)),
                  pl.BlockSpec(hc_np.shape, lambda b: (0, 0))],
        out_specs=pl.BlockSpec((tb, ho, wo), lambda b: (b, 0, 0)),
        compiler_params=pltpu.CompilerParams(
            dimension_semantics=("parallel",),
            vmem_limit_bytes=64 * 1024 * 1024),
        cost_estimate=pl.CostEstimate(
            flops=flops, transcendentals=0, bytes_accessed=bytes_accessed),
    )(xb, jnp.asarray(wct_np), jnp.asarray(hc_np))
    return out[:B].reshape(N, C, ho, wo)
